# baseline (speedup 1.0000x reference)
import functools

import jax
import jax.numpy as jnp
import numpy as np
from jax import lax
from jax.experimental import pallas as pl
from jax.experimental.pallas import tpu as pltpu
from jax.experimental.pallas import tpu_sc as plsc

_D = 128
_SCALE = float(np.sqrt(_D))
_NC, _NS, _L = 2, 16, 16
_NW = _NC * _NS


def _pos_table(seq_len: int) -> jax.Array:
    pos = jnp.arange(seq_len, dtype=jnp.float32)[:, None]
    i2 = jnp.arange(0, _D, 2, dtype=jnp.float32)
    ang = pos / jnp.power(10000.0, i2 / float(_D))
    enc = jnp.zeros((seq_len, _D), dtype=jnp.float32)
    enc = enc.at[:, 0::2].set(jnp.sin(ang))
    enc = enc.at[:, 1::2].set(jnp.cos(ang))
    return enc


def kernel(x, table):
    B, S = x.shape
    N = B * S
    b_per_w = N // _NW
    NCH = 4
    C = b_per_w // NCH
    assert N % _NW == 0 and b_per_w % NCH == 0 and S % b_per_w == 0

    pos_div = _pos_table(S) * np.float32(1.0 / _SCALE)
    xw = x.reshape(_NW, NCH, C)

    mesh = plsc.VectorSubcoreMesh(
        core_axis_name="c", subcore_axis_name="s",
        num_cores=_NC, num_subcores=_NS,
    )

    @functools.partial(
        pl.kernel,
        out_type=jax.ShapeDtypeStruct((N, _D), jnp.float32),
        mesh=mesh,
        scratch_types=[
            pltpu.VMEM((NCH, C), jnp.int32),
            pltpu.VMEM((NCH, C, _D), jnp.float32),
            [pltpu.SemaphoreType.DMA] * NCH,
            [pltpu.SemaphoreType.DMA] * NCH,
            [pltpu.SemaphoreType.DMA] * NCH,
        ],
    )
    def emb_kernel(x_hbm, table_hbm, pos_hbm, out_hbm,
                   idx_v, rows_v, psems, gsems, wsems):
        wid = lax.axis_index("s") * _NC + lax.axis_index("c")
        base = wid * b_per_w
        pos_base = lax.rem(base, S)

        pltpu.sync_copy(x_hbm.at[wid], idx_v)
        pos_descs = [
            pltpu.async_copy(pos_hbm.at[pl.ds(pos_base + c * C, C)],
                             rows_v.at[c], psems[c])
            for c in range(NCH)
        ]
        gadd_descs = []
        for c in range(NCH):
            pos_descs[c].wait()
            gadd_descs.append(
                pltpu.async_copy(table_hbm.at[idx_v.at[c]], rows_v.at[c],
                                 gsems[c], add=True))
        wb_descs = []
        for c in range(NCH):
            gadd_descs[c].wait()

            def body(i, _, c=c):
                for j in range(_D // _L):
                    sl = pl.ds(j * _L, _L)
                    rows_v[c, i, sl] = rows_v[c, i, sl] * _SCALE
                return 0

            lax.fori_loop(0, C, body, 0)
            wb_descs.append(
                pltpu.async_copy(rows_v.at[c],
                                 out_hbm.at[pl.ds(base + c * C, C)],
                                 wsems[c]))
        for c in range(NCH):
            wb_descs[c].wait()

    out = emb_kernel(xw, table, pos_div)
    return out.reshape(B, S, _D)


# speedup vs baseline: 1.2512x; 1.2512x over previous
import functools

import jax
import jax.numpy as jnp
import numpy as np
from jax import lax
from jax.experimental import pallas as pl
from jax.experimental.pallas import tpu as pltpu
from jax.experimental.pallas import tpu_sc as plsc

_D = 128
_SCALE = float(np.sqrt(_D))
_NC, _NS, _L = 2, 16, 16
_NW = _NC * _NS


def _pos_table(seq_len: int) -> np.ndarray:
    pos = np.arange(seq_len, dtype=np.float32)[:, None]
    i2 = np.arange(0, _D, 2, dtype=np.float32)
    ang = (pos / np.power(np.float32(10000.0), i2 / np.float32(_D))).astype(np.float32)
    enc = np.zeros((seq_len, _D), dtype=np.float32)
    enc[:, 0::2] = np.sin(ang)
    enc[:, 1::2] = np.cos(ang)
    return enc


def kernel(x, table):
    B, S = x.shape
    N = B * S
    b_per_w = N // _NW
    NCH = 4
    C = b_per_w // NCH
    assert N % _NW == 0 and b_per_w % NCH == 0 and S % b_per_w == 0

    pos_div = _pos_table(S) * np.float32(1.0 / _SCALE)
    xw = x.reshape(_NW, NCH, C)

    mesh = plsc.VectorSubcoreMesh(
        core_axis_name="c", subcore_axis_name="s",
        num_cores=_NC, num_subcores=_NS,
    )

    @functools.partial(
        pl.kernel,
        out_type=jax.ShapeDtypeStruct((N, _D), jnp.float32),
        mesh=mesh,
        scratch_types=[
            pltpu.VMEM((NCH, C), jnp.int32),
            pltpu.VMEM((NCH, C, _D), jnp.float32),
            [pltpu.SemaphoreType.DMA] * NCH,
            [pltpu.SemaphoreType.DMA] * NCH,
            [pltpu.SemaphoreType.DMA] * NCH,
        ],
    )
    def emb_kernel(x_hbm, table_hbm, pos_hbm, out_hbm,
                   idx_v, rows_v, psems, gsems, wsems):
        wid = lax.axis_index("s") * _NC + lax.axis_index("c")
        base = wid * b_per_w
        pos_base = lax.rem(base, S)

        pltpu.sync_copy(x_hbm.at[wid], idx_v)
        pos_descs = [
            pltpu.async_copy(pos_hbm.at[pl.ds(pos_base + c * C, C)],
                             rows_v.at[c], psems[c])
            for c in range(NCH)
        ]
        gadd_descs = []
        for c in range(NCH):
            pos_descs[c].wait()
            gadd_descs.append(
                pltpu.async_copy(table_hbm.at[idx_v.at[c]], rows_v.at[c],
                                 gsems[c], add=True))
        wb_descs = []
        for c in range(NCH):
            gadd_descs[c].wait()

            def body(i, _, c=c):
                for j in range(_D // _L):
                    sl = pl.ds(j * _L, _L)
                    rows_v[c, i, sl] = rows_v[c, i, sl] * _SCALE
                return 0

            lax.fori_loop(0, C, body, 0)
            wb_descs.append(
                pltpu.async_copy(rows_v.at[c],
                                 out_hbm.at[pl.ds(base + c * C, C)],
                                 wsems[c]))
        for c in range(NCH):
            wb_descs[c].wait()

    out = emb_kernel(xw, table, pos_div)
    return out.reshape(B, S, _D)
